# baseline (speedup 1.0000x reference)
import jax
import jax.numpy as jnp
from jax.experimental import pallas as pl
from jax.experimental.pallas import tpu as pltpu

_B = 64
_D = 1024
_HD = _D // 2
_FF = 2048
_NG = 4
_EPG = 4
_NE = _NG * _EPG
_FB = 2
_FBS = _FF // _FB


def _layernorm(y, g, b):
    m = jnp.mean(y, axis=-1, keepdims=True)
    v = jnp.mean((y - m) ** 2, axis=-1, keepdims=True)
    return (y - m) * jax.lax.rsqrt(v + 1e-5) * g + b


def _body(x_ref, wv_ref, bv_ref, wo_ref, bo_ref, ln1g_ref, ln1b_ref,
          ggw_ref, gew_ref, b2_ref, w1_ref, b1_ref, w2_ref,
          ln2g_ref, ln2b_ref, out_ref, vbuf, abuf, x1s, combs, acc_ref):
    e = pl.program_id(0)
    f = pl.program_id(1)

    @pl.when(jnp.logical_and(e == 0, f == 0))
    def _v_half0():
        vbuf[:, :_HD] = jax.lax.dot_general(
            x_ref[...], wv_ref[0], (((1,), (1,)), ((), ())),
            preferred_element_type=jnp.float32) + bv_ref[:, :_HD]

    @pl.when(jnp.logical_and(e == 0, f == 1))
    def _v_half1():
        vbuf[:, _HD:] = jax.lax.dot_general(
            x_ref[...], wv_ref[0], (((1,), (1,)), ((), ())),
            preferred_element_type=jnp.float32) + bv_ref[:, _HD:]

    @pl.when(jnp.logical_and(e == 1, f == 0))
    def _attn_half0():
        abuf[:, :_HD] = jax.lax.dot_general(
            vbuf[...], wo_ref[0], (((1,), (1,)), ((), ())),
            preferred_element_type=jnp.float32)

    @pl.when(jnp.logical_and(e == 1, f == 1))
    def _attn_half1_ln_gate():
        attn1 = jax.lax.dot_general(
            vbuf[...], wo_ref[0], (((1,), (1,)), ((), ())),
            preferred_element_type=jnp.float32)
        attn = jnp.concatenate([abuf[:, :_HD], attn1], axis=1) + bo_ref[...]
        x1 = _layernorm(x_ref[...] + attn, ln1g_ref[...], ln1b_ref[...])
        x1s[...] = x1
        gl = jnp.dot(x1, ggw_ref[...], preferred_element_type=jnp.float32)
        gexp = jnp.exp(gl - jnp.max(gl, axis=-1, keepdims=True))
        gp = gexp / jnp.sum(gexp, axis=-1, keepdims=True)
        el = jnp.dot(x1, gew_ref[...], preferred_element_type=jnp.float32)
        eexp = jnp.exp(el)
        gi = jax.lax.broadcasted_iota(jnp.int32, (_NE, _NE), 0) // _EPG
        gj = jax.lax.broadcasted_iota(jnp.int32, (_NE, _NE), 1) // _EPG
        gsum = jnp.where(gi == gj, 1.0, 0.0)
        denom = jnp.dot(eexp, gsum, preferred_element_type=jnp.float32)
        ep = eexp / denom
        ri = jax.lax.broadcasted_iota(jnp.int32, (_NG, _NE), 0)
        rj = jax.lax.broadcasted_iota(jnp.int32, (_NG, _NE), 1) // _EPG
        rep = jnp.where(ri == rj, 1.0, 0.0)
        comb = jnp.dot(gp, rep, preferred_element_type=jnp.float32) * ep
        combs[...] = comb
        acc_ref[...] = jnp.dot(comb, b2_ref[...],
                               preferred_element_type=jnp.float32)

    @pl.when(e >= 2)
    def _expert():
        ee = e - 2
        x1 = x1s[...].astype(jnp.bfloat16)
        h32 = jax.lax.dot_general(
            x1, w1_ref[0].astype(jnp.bfloat16), (((1,), (0,)), ((), ())),
            preferred_element_type=jnp.float32) + b1_ref[0]
        h = jnp.maximum(h32, 0.0).astype(jnp.bfloat16)
        p = jax.lax.dot_general(
            h, w2_ref[0].astype(jnp.bfloat16), (((1,), (0,)), ((), ())),
            preferred_element_type=jnp.float32)
        lane = jax.lax.broadcasted_iota(jnp.int32, (_B, _NE), 1)
        c = jnp.sum(jnp.where(lane == ee, combs[...], 0.0), axis=1,
                    keepdims=True)
        acc_ref[...] += c * p

        @pl.when(jnp.logical_and(e == _NE + 1, f == _FB - 1))
        def _finish():
            out_ref[...] = _layernorm(x1s[...] + acc_ref[...],
                                      ln2g_ref[...], ln2b_ref[...])


def kernel(x, in_proj_w, in_proj_b, out_proj_w, out_proj_b, gate_group_w,
           gate_expert_w, w1, b1, w2, b2, ln1_g, ln1_b, ln2_g, ln2_b):
    Bq, Sq, D = x.shape
    x2d = x.reshape(_B, _D)
    wvr = in_proj_w[2 * _D:].reshape(2, _HD, _D)
    bv = in_proj_b[2 * _D:].reshape(1, _D)
    wor = out_proj_w.reshape(2, _HD, _D)
    bo = out_proj_b.reshape(1, _D)
    gew = gate_expert_w.transpose(1, 0, 2).reshape(_D, _NE)
    w1r = w1.reshape(_NE, _D, _FF)
    b1r = b1.reshape(_NE, 1, _FF)
    w2r = w2.reshape(_NE, _FF, _D)
    b2r = b2.reshape(_NE, _D)

    const = lambda e, f: (0, 0)
    i32 = jnp.int32

    def wv_map(e, f):
        return (jnp.where(e == 0, f, 1).astype(i32), 0, 0)

    def wo_map(e, f):
        return (jnp.where(e <= 0, 0, jnp.where(e == 1, f, 1)).astype(i32),
                0, 0)

    def w1_map(e, f):
        ee = jnp.maximum(e - 2, 0).astype(i32)
        return (ee, 0, jnp.where(e < 2, 0, f).astype(i32))

    def b1_map(e, f):
        ee = jnp.maximum(e - 2, 0).astype(i32)
        return (ee, 0, jnp.where(e < 2, 0, f).astype(i32))

    def w2_map(e, f):
        ee = jnp.maximum(e - 2, 0).astype(i32)
        return (ee, jnp.where(e < 2, 0, f).astype(i32), 0)

    out = pl.pallas_call(
        _body,
        grid=(_NE + 2, _FB),
        in_specs=[
            pl.BlockSpec((_B, _D), const),
            pl.BlockSpec((1, _HD, _D), wv_map),
            pl.BlockSpec((1, _D), const),
            pl.BlockSpec((1, _HD, _D), wo_map),
            pl.BlockSpec((1, _D), const),
            pl.BlockSpec((1, _D), const),
            pl.BlockSpec((1, _D), const),
            pl.BlockSpec((_D, _NG), const),
            pl.BlockSpec((_D, _NE), const),
            pl.BlockSpec((_NE, _D), const),
            pl.BlockSpec((1, _D, _FBS), w1_map),
            pl.BlockSpec((1, 1, _FBS), b1_map),
            pl.BlockSpec((1, _FBS, _D), w2_map),
            pl.BlockSpec((1, _D), const),
            pl.BlockSpec((1, _D), const),
        ],
        out_specs=pl.BlockSpec((_B, _D), const),
        out_shape=jax.ShapeDtypeStruct((_B, _D), jnp.float32),
        scratch_shapes=[
            pltpu.VMEM((_B, _D), jnp.float32),
            pltpu.VMEM((_B, _D), jnp.float32),
            pltpu.VMEM((_B, _D), jnp.float32),
            pltpu.VMEM((_B, _NE), jnp.float32),
            pltpu.VMEM((_B, _D), jnp.float32),
        ],
        compiler_params=pltpu.CompilerParams(
            dimension_semantics=("arbitrary", "arbitrary"),
        ),
    )(x2d, wvr, bv, wor, bo, ln1_g.reshape(1, _D), ln1_b.reshape(1, _D),
      gate_group_w, gew, b2r, w1r, b1r, w2r, ln2_g.reshape(1, _D),
      ln2_b.reshape(1, _D))

    return out.reshape(Bq, Sq, D)


# speedup vs baseline: 1.0095x; 1.0095x over previous
import jax
import jax.numpy as jnp
from jax.experimental import pallas as pl
from jax.experimental.pallas import tpu as pltpu

_B = 64
_D = 1024
_FF = 2048
_NG = 4
_EPG = 4
_NE = _NG * _EPG
_FB = 2
_FBS = _FF // _FB


def _layernorm(y, g, b):
    m = jnp.mean(y, axis=-1, keepdims=True)
    v = jnp.mean((y - m) ** 2, axis=-1, keepdims=True)
    return (y - m) * jax.lax.rsqrt(v + 1e-5) * g + b


def _prologue_body(x_ref, wv_ref, bv_ref, wo_ref, bo_ref, ln1g_ref, ln1b_ref,
                   ggw_ref, gew_ref, b2_ref, x1_ref, comb_ref, seed_ref):
    x = x_ref[...]
    v = jax.lax.dot_general(x, wv_ref[...], (((1,), (1,)), ((), ())),
                            preferred_element_type=jnp.float32) + bv_ref[...]
    attn = jax.lax.dot_general(v, wo_ref[...], (((1,), (1,)), ((), ())),
                               preferred_element_type=jnp.float32) + bo_ref[...]
    x1 = _layernorm(x + attn, ln1g_ref[...], ln1b_ref[...])
    x1_ref[...] = x1
    gl = jnp.dot(x1, ggw_ref[...], preferred_element_type=jnp.float32)
    gexp = jnp.exp(gl - jnp.max(gl, axis=-1, keepdims=True))
    gp = gexp / jnp.sum(gexp, axis=-1, keepdims=True)
    el = jnp.dot(x1, gew_ref[...], preferred_element_type=jnp.float32)
    eexp = jnp.exp(el)
    gi = jax.lax.broadcasted_iota(jnp.int32, (_NE, _NE), 0) // _EPG
    gj = jax.lax.broadcasted_iota(jnp.int32, (_NE, _NE), 1) // _EPG
    gsum_mat = jnp.where(gi == gj, 1.0, 0.0)
    denom = jnp.dot(eexp, gsum_mat, preferred_element_type=jnp.float32)
    ep = eexp / denom
    ri = jax.lax.broadcasted_iota(jnp.int32, (_NG, _NE), 0)
    rj = jax.lax.broadcasted_iota(jnp.int32, (_NG, _NE), 1) // _EPG
    rep = jnp.where(ri == rj, 1.0, 0.0)
    gp_full = jnp.dot(gp, rep, preferred_element_type=jnp.float32)
    comb = gp_full * ep
    comb_ref[...] = comb
    seed_ref[...] = jnp.dot(comb, b2_ref[...],
                            preferred_element_type=jnp.float32)


def _moe_body(x1_ref, comb_ref, seed_ref, w1_ref, b1_ref, w2_ref,
              ln2g_ref, ln2b_ref, out_ref, acc_ref):
    e = pl.program_id(0)
    f = pl.program_id(1)

    @pl.when(jnp.logical_and(e == 0, f == 0))
    def _init():
        acc_ref[...] = seed_ref[...]

    x1 = x1_ref[...].astype(jnp.bfloat16)
    h32 = jax.lax.dot_general(
        x1, w1_ref[0].astype(jnp.bfloat16), (((1,), (0,)), ((), ())),
        preferred_element_type=jnp.float32) + b1_ref[0]
    h = jnp.maximum(h32, 0.0).astype(jnp.bfloat16)
    p = jax.lax.dot_general(
        h, w2_ref[0].astype(jnp.bfloat16), (((1,), (0,)), ((), ())),
        preferred_element_type=jnp.float32)
    lane = jax.lax.broadcasted_iota(jnp.int32, (_B, _NE), 1)
    c = jnp.sum(jnp.where(lane == e, comb_ref[...], 0.0), axis=1,
                keepdims=True)
    acc_ref[...] += c * p

    @pl.when(jnp.logical_and(e == _NE - 1, f == _FB - 1))
    def _finish():
        out_ref[...] = _layernorm(x1_ref[...] + acc_ref[...], ln2g_ref[...],
                                  ln2b_ref[...])


def kernel(x, in_proj_w, in_proj_b, out_proj_w, out_proj_b, gate_group_w,
           gate_expert_w, w1, b1, w2, b2, ln1_g, ln1_b, ln2_g, ln2_b):
    Bq, Sq, D = x.shape
    x2d = x.reshape(_B, _D)
    wv = in_proj_w[2 * _D:]
    bv = in_proj_b[2 * _D:].reshape(1, _D)
    bo = out_proj_b.reshape(1, _D)
    gew = gate_expert_w.transpose(1, 0, 2).reshape(_D, _NE)

    x1, comb, seed = pl.pallas_call(
        _prologue_body,
        out_shape=(
            jax.ShapeDtypeStruct((_B, _D), jnp.float32),
            jax.ShapeDtypeStruct((_B, _NE), jnp.float32),
            jax.ShapeDtypeStruct((_B, _D), jnp.float32),
        ),
    )(x2d, wv, bv, out_proj_w, bo, ln1_g.reshape(1, _D),
      ln1_b.reshape(1, _D), gate_group_w, gew, b2.reshape(_NE, _D))

    w1r = w1.reshape(_NE, _D, _FF)
    b1r = b1.reshape(_NE, 1, _FF)
    w2r = w2.reshape(_NE, _FF, _D)

    out = pl.pallas_call(
        _moe_body,
        grid=(_NE, _FB),
        in_specs=[
            pl.BlockSpec((_B, _D), lambda e, f: (0, 0)),
            pl.BlockSpec((_B, _NE), lambda e, f: (0, 0)),
            pl.BlockSpec((_B, _D), lambda e, f: (0, 0)),
            pl.BlockSpec((1, _D, _FBS), lambda e, f: (e, 0, f)),
            pl.BlockSpec((1, 1, _FBS), lambda e, f: (e, 0, f)),
            pl.BlockSpec((1, _FBS, _D), lambda e, f: (e, f, 0)),
            pl.BlockSpec((1, _D), lambda e, f: (0, 0)),
            pl.BlockSpec((1, _D), lambda e, f: (0, 0)),
        ],
        out_specs=pl.BlockSpec((_B, _D), lambda e, f: (0, 0)),
        out_shape=jax.ShapeDtypeStruct((_B, _D), jnp.float32),
        scratch_shapes=[pltpu.VMEM((_B, _D), jnp.float32)],
        compiler_params=pltpu.CompilerParams(
            dimension_semantics=("arbitrary", "arbitrary"),
        ),
    )(x1, comb, seed, w1r, b1r, w2r, ln2_g.reshape(1, _D),
      ln2_b.reshape(1, _D))

    return out.reshape(Bq, Sq, D)
